# baseline (speedup 1.0000x reference)
import functools

import jax
import jax.numpy as jnp
from jax import lax
from jax.experimental import pallas as pl
from jax.experimental.pallas import tpu as pltpu
from jax.experimental.pallas import tpu_sc as plsc

K = 1024
DM = 64
NTOK = 32768
TILE = 1024
NT = NTOK // TILE
LOSS_SCALE = 0.25 / (NTOK * DM)


def _vq_body(x_ref, eTm2_ref, esq_ref, cs_ref,
             idx_ref, counts_ref, stats_ref):
    pid = pl.program_id(0)
    x = x_ref[...]

    xsq = jnp.sum(x * x, axis=1, keepdims=True)
    mm2 = jnp.dot(x, eTm2_ref[...], preferred_element_type=jnp.float32)
    d2 = (xsq + mm2) + esq_ref[...]
    dist = jnp.sqrt(jnp.maximum(d2, 0.0))
    dmin = jnp.min(dist, axis=1, keepdims=True)
    kiota = jax.lax.broadcasted_iota(jnp.int32, (TILE, K), 1)
    idxv = jnp.min(jnp.where(dist == dmin, kiota, K), axis=1)
    idxv = idxv.astype(jnp.int32).reshape(TILE, 1)
    idx_ref[...] = idxv
    onehot = (kiota == idxv).astype(jnp.float32)
    ones_row = jnp.ones((1, TILE), jnp.float32)
    counts_add = jnp.dot(ones_row, onehot,
                         preferred_element_type=jnp.float32)
    loss_t = jnp.sum(dmin * dmin)
    lane = jax.lax.broadcasted_iota(jnp.int32, (1, 128), 1)
    stat_add = jnp.where(lane == 0, loss_t, 0.0)

    @pl.when(pid == 0)
    def _init():
        counts_ref[...] = counts_add
        stats_ref[...] = stat_add

    @pl.when(pid > 0)
    def _accum():
        counts_ref[...] += counts_add
        stats_ref[...] += stat_add

    @pl.when(pid == NT - 1)
    def _finish():
        counts = counts_ref[...]
        avg = counts * (1.0 / NTOK)
        ent = jnp.sum(avg * jnp.log(avg + 1e-10))
        perp = jnp.exp(-ent)
        cs = cs_ref[...]
        used = jnp.sum((cs > 1e-5).astype(jnp.float32)) * (1.0 / K)
        s = stats_ref[...]
        loss_total = jnp.sum(jnp.where(lane == 0, s, 0.0)) * LOSS_SCALE
        stats_ref[...] = jnp.where(lane == 0, loss_total,
                         jnp.where(lane == 1, perp,
                         jnp.where(lane == 2, used, 0.0)))


def _vq_call(flat, eTm2, esq, cs2, interpret=False):
    return pl.pallas_call(
        _vq_body,
        grid=(NT,),
        in_specs=[
            pl.BlockSpec((TILE, DM), lambda i: (i, 0)),
            pl.BlockSpec((DM, K), lambda i: (0, 0)),
            pl.BlockSpec((1, K), lambda i: (0, 0)),
            pl.BlockSpec((8, 128), lambda i: (0, 0)),
        ],
        out_specs=[
            pl.BlockSpec((TILE, 1), lambda i: (i, 0)),
            pl.BlockSpec((1, K), lambda i: (0, 0)),
            pl.BlockSpec((1, 128), lambda i: (0, 0)),
        ],
        out_shape=[
            jax.ShapeDtypeStruct((NTOK, 1), jnp.int32),
            jax.ShapeDtypeStruct((1, K), jnp.float32),
            jax.ShapeDtypeStruct((1, 128), jnp.float32),
        ],
        interpret=interpret,
    )(flat, eTm2, esq, cs2)


DP = 128


def _sc_gather(table_padded, idx):
    info = plsc.get_sparse_core_info()
    nw = info.num_cores * info.num_subcores
    bpw = NTOK // nw
    mesh = plsc.VectorSubcoreMesh(core_axis_name="c", subcore_axis_name="s")

    nch = 2
    ch = bpw // nch

    @functools.partial(
        pl.kernel, mesh=mesh,
        out_type=jax.ShapeDtypeStruct((NTOK, DP), jnp.float32),
        scratch_types=[
            pltpu.VMEM((bpw,), jnp.int32),
            pltpu.VMEM((ch, DP), jnp.float32),
            pltpu.SemaphoreType.DMA,
        ],
    )
    def k(table_hbm, idx_hbm, out_hbm, idx_v, rows_v, sem):
        wid = lax.axis_index("s") * info.num_cores + lax.axis_index("c")
        base = wid * bpw
        pltpu.sync_copy(idx_hbm.at[pl.ds(base, bpw)], idx_v)
        for c in range(nch):
            pltpu.async_copy(table_hbm.at[idx_v.at[pl.ds(c * ch, ch)]],
                             rows_v, sem).wait()
            pltpu.sync_copy(rows_v, out_hbm.at[pl.ds(base + c * ch, ch)])

    return k(table_padded, idx)


def kernel(z, embedding, cluster_size):
    B, C, D, H, W = z.shape
    flat = jnp.transpose(z, (0, 2, 3, 4, 1)).reshape(NTOK, DM)
    embedding = embedding.at[0].set(0.0).at[1].set(6.0)
    eTm2 = embedding.T * (-2.0)
    esq = jnp.sum(embedding ** 2, axis=1)[None, :]
    cs2 = cluster_size.reshape(8, 128)
    idx, _counts, stats = _vq_call(flat, eTm2, esq, cs2)
    table_padded = jnp.pad(embedding, ((0, 0), (0, DP - DM)))
    q = _sc_gather(table_padded, idx.reshape(NTOK))[:, :DM]
    qr = jnp.transpose(q.reshape(B, D, H, W, C), (0, 4, 1, 2, 3))
    idx_out = idx.reshape(B, D, H, W)
    total_loss = stats[0, 0]
    perplexity = stats[0, 1]
    used = stats[0, 2]
    return (qr, total_loss, idx_out, perplexity, used)
